# baseline (speedup 1.0000x reference)
import functools

import jax
import jax.numpy as jnp
from jax import lax
from jax.experimental import pallas as pl
from jax.experimental.pallas import tpu as pltpu
from jax.experimental.pallas import tpu_sc as plsc

EMBEDDING_DIM = 64
NUM_TOKENS = 4096 * 200
NUM_WORKERS = 32
ROWS_PER_WORKER = NUM_TOKENS // NUM_WORKERS
CHUNK = 512
NUM_CHUNKS = ROWS_PER_WORKER // CHUNK

_MESH = plsc.VectorSubcoreMesh(core_axis_name="c", subcore_axis_name="s")


@functools.partial(
    pl.kernel,
    mesh=_MESH,
    out_type=jax.ShapeDtypeStruct((NUM_TOKENS, EMBEDDING_DIM), jnp.float32),
    scratch_types=[
        pltpu.VMEM((CHUNK,), jnp.int32),
        pltpu.VMEM((CHUNK, EMBEDDING_DIM), jnp.float32),
        pltpu.SemaphoreType.DMA,
    ],
    compiler_params=pltpu.CompilerParams(use_tc_tiling_on_sc=False),
)
def _gather_kernel(tok_hbm, table_hbm, out_hbm, idx_v, rows_v, sem):
    wid = lax.axis_index("s") * 2 + lax.axis_index("c")
    base = wid * ROWS_PER_WORKER

    def body(i, carry):
        off = base + i * CHUNK
        pltpu.sync_copy(tok_hbm.at[pl.ds(off, CHUNK)], idx_v)
        pltpu.async_copy(table_hbm.at[idx_v], rows_v, sem).wait()
        pltpu.sync_copy(rows_v, out_hbm.at[pl.ds(off, CHUNK)])
        return carry

    lax.fori_loop(0, NUM_CHUNKS, body, 0, unroll=False)


def kernel(token_ids, weight):
    flat = token_ids.reshape(-1)
    out = _gather_kernel(flat, weight)
    return out.reshape(token_ids.shape + (EMBEDDING_DIM,))


# speedup vs baseline: 1.0414x; 1.0414x over previous
import functools

import jax
import jax.numpy as jnp
from jax import lax
from jax.experimental import pallas as pl
from jax.experimental.pallas import tpu as pltpu
from jax.experimental.pallas import tpu_sc as plsc

EMBEDDING_DIM = 64
NUM_TOKENS = 4096 * 200
NUM_WORKERS = 32
ROWS_PER_WORKER = NUM_TOKENS // NUM_WORKERS
CHUNK = 320
NUM_CHUNKS = ROWS_PER_WORKER // CHUNK
NBUF = 4
NUM_GROUPS = NUM_CHUNKS // NBUF

_MESH = plsc.VectorSubcoreMesh(core_axis_name="c", subcore_axis_name="s")


@functools.partial(
    pl.kernel,
    mesh=_MESH,
    out_type=jax.ShapeDtypeStruct((NUM_TOKENS, EMBEDDING_DIM), jnp.float32),
    scratch_types=[
        pltpu.VMEM((ROWS_PER_WORKER,), jnp.int32),
        [pltpu.VMEM((CHUNK, EMBEDDING_DIM), jnp.float32) for _ in range(NBUF)],
        [pltpu.SemaphoreType.DMA for _ in range(NBUF)],
        [pltpu.SemaphoreType.DMA for _ in range(NBUF)],
    ],
    compiler_params=pltpu.CompilerParams(use_tc_tiling_on_sc=False),
)
def _gather_kernel(tok_hbm, table_hbm, out_hbm, idx_v, rows, gsems, ssems):
    wid = lax.axis_index("s") * 2 + lax.axis_index("c")
    base = wid * ROWS_PER_WORKER

    pltpu.sync_copy(tok_hbm.at[pl.ds(base, ROWS_PER_WORKER)], idx_v)

    def fire_gather(c, slot):
        pltpu.async_copy(
            table_hbm.at[idx_v.at[pl.ds(c * CHUNK, CHUNK)]], rows[slot],
            gsems[slot],
        )

    def wait_gather(slot):
        pltpu.make_async_copy(
            table_hbm.at[pl.ds(0, CHUNK)], rows[slot], gsems[slot]
        ).wait()

    def fire_store(c, slot):
        pltpu.async_copy(
            rows[slot], out_hbm.at[pl.ds(base + c * CHUNK, CHUNK)], ssems[slot]
        )

    def wait_store(slot):
        pltpu.make_async_copy(
            rows[slot], out_hbm.at[pl.ds(base, CHUNK)], ssems[slot]
        ).wait()

    fire_gather(0, 0)

    def body(h, carry):
        for k in range(NBUF):
            c = h * NBUF + k
            kn = (k + 1) % NBUF

            @pl.when(c >= NBUF - 1)
            def _():
                wait_store(kn)

            @pl.when(c + 1 < NUM_CHUNKS)
            def _():
                fire_gather(c + 1, kn)

            wait_gather(k)
            fire_store(c, k)
        return carry

    lax.fori_loop(0, NUM_GROUPS, body, 0, unroll=False)

    for k in range(1, NBUF):
        wait_store(k)


def kernel(token_ids, weight):
    flat = token_ids.reshape(-1)
    out = _gather_kernel(flat, weight)
    return out.reshape(token_ids.shape + (EMBEDDING_DIM,))


# speedup vs baseline: 1.2686x; 1.2182x over previous
import functools

import jax
import jax.numpy as jnp
from jax import lax
from jax.experimental import pallas as pl
from jax.experimental.pallas import tpu as pltpu
from jax.experimental.pallas import tpu_sc as plsc

EMBEDDING_DIM = 64
PADDED_DIM = 128
NUM_TOKENS = 4096 * 200
NUM_WORKERS = 32
ROWS_PER_WORKER = NUM_TOKENS // NUM_WORKERS
CHUNK = 200
NUM_CHUNKS = ROWS_PER_WORKER // CHUNK
LANES = 16

_MESH = plsc.VectorSubcoreMesh(core_axis_name="c", subcore_axis_name="s")


@functools.partial(
    pl.kernel,
    mesh=_MESH,
    out_type=jax.ShapeDtypeStruct((NUM_TOKENS // 8, 8, EMBEDDING_DIM), jnp.float32),
    scratch_types=[
        pltpu.VMEM((ROWS_PER_WORKER,), jnp.int32),
        [pltpu.VMEM((CHUNK, PADDED_DIM), jnp.float32) for _ in range(2)],
        [pltpu.VMEM((CHUNK // 8, 8, EMBEDDING_DIM), jnp.float32) for _ in range(2)],
        [pltpu.SemaphoreType.DMA for _ in range(2)],
        [pltpu.SemaphoreType.DMA for _ in range(2)],
    ],
    compiler_params=pltpu.CompilerParams(use_tc_tiling_on_sc=True),
)
def _gather_kernel(tok_hbm, table_hbm, out_hbm, idx_v, rows, cbufs, gsems, ssems):
    wid = lax.axis_index("s") * 2 + lax.axis_index("c")
    base = wid * ROWS_PER_WORKER

    pltpu.sync_copy(tok_hbm.at[pl.ds(base, ROWS_PER_WORKER)], idx_v)

    def fire_gather(c, slot):
        pltpu.async_copy(
            table_hbm.at[idx_v.at[pl.ds(c * CHUNK, CHUNK)]], rows[slot],
            gsems[slot],
        )

    def wait_gather(slot):
        pltpu.make_async_copy(
            table_hbm.at[pl.ds(0, CHUNK)], rows[slot], gsems[slot]
        ).wait()

    def fire_store(c, slot):
        pltpu.async_copy(
            cbufs[slot],
            out_hbm.at[pl.ds((base + c * CHUNK) // 8, CHUNK // 8)],
            ssems[slot],
        )

    def wait_store(slot):
        pltpu.make_async_copy(
            cbufs[slot], out_hbm.at[pl.ds(0, CHUNK // 8)], ssems[slot]
        ).wait()

    def extract(slot):
        def tile_body(q, carry):
            for r in range(8):
                for g in range(4):
                    vals = rows[slot][q * 8 + r, pl.ds(g * LANES, LANES)]
                    cbufs[slot][q, r, pl.ds(g * LANES, LANES)] = vals
            return carry

        lax.fori_loop(0, CHUNK // 8, tile_body, 0, unroll=False)

    fire_gather(0, 0)

    def body(h, carry):
        for k in range(2):
            c = h * 2 + k
            kn = (k + 1) % 2

            @pl.when(c + 1 < NUM_CHUNKS)
            def _():
                fire_gather(c + 1, kn)

            wait_gather(k)

            @pl.when(c >= 2)
            def _():
                wait_store(k)

            extract(k)
            fire_store(c, k)
        return carry

    lax.fori_loop(0, NUM_CHUNKS // 2, body, 0, unroll=False)

    for k in range(2):
        wait_store(k)


def kernel(token_ids, weight):
    flat = token_ids.reshape(-1)
    padded = jnp.pad(weight, ((0, 0), (0, PADDED_DIM - EMBEDDING_DIM)))
    out = _gather_kernel(flat, padded)
    return out.reshape(token_ids.shape + (EMBEDDING_DIM,))
